# baseline (speedup 1.0000x reference)
import jax
import jax.numpy as jnp
from jax.experimental import pallas as pl
from jax.experimental.pallas import tpu as pltpu


def _add_kernel(x_ref, t_ref, o_ref):
    o_ref[...] = x_ref[...] + t_ref[...]


def kernel(encoded_patches, pos_table):
    B, P, D = encoded_patches.shape
    PP = 192
    grid = (B, P // PP)
    return pl.pallas_call(
        _add_kernel,
        grid=grid,
        in_specs=[
            pl.BlockSpec((1, PP, D), lambda b, p: (b, p, 0)),
            pl.BlockSpec((PP, D), lambda b, p: (p, 0)),
        ],
        out_specs=pl.BlockSpec((1, PP, D), lambda b, p: (b, p, 0)),
        out_shape=jax.ShapeDtypeStruct((B, P, D), encoded_patches.dtype),
        compiler_params=pltpu.CompilerParams(
            dimension_semantics=("parallel", "parallel"),
        ),
    )(encoded_patches, pos_table)


# speedup vs baseline: 1.1635x; 1.1635x over previous
import jax
import jax.numpy as jnp
from jax.experimental import pallas as pl
from jax.experimental.pallas import tpu as pltpu


def _add_kernel(x_ref, t_ref, o_ref):
    o_ref[...] = x_ref[...] + t_ref[...]


def kernel(encoded_patches, pos_table):
    B, P, D = encoded_patches.shape
    PP = 192
    grid = (P // PP, B)
    return pl.pallas_call(
        _add_kernel,
        grid=grid,
        in_specs=[
            pl.BlockSpec((1, PP, D), lambda p, b: (b, p, 0)),
            pl.BlockSpec((PP, D), lambda p, b: (p, 0)),
        ],
        out_specs=pl.BlockSpec((1, PP, D), lambda p, b: (b, p, 0)),
        out_shape=jax.ShapeDtypeStruct((B, P, D), encoded_patches.dtype),
        compiler_params=pltpu.CompilerParams(
            dimension_semantics=("parallel", "parallel"),
        ),
    )(encoded_patches, pos_table)


# speedup vs baseline: 2.3366x; 2.0082x over previous
import jax
import jax.numpy as jnp
from jax.experimental import pallas as pl
from jax.experimental.pallas import tpu as pltpu


def _add_kernel(x_ref, t_ref, o_ref):
    o_ref[...] = x_ref[...] + t_ref[...]


def kernel(encoded_patches, pos_table):
    B, P, D = encoded_patches.shape
    BB = 2
    grid = (B // BB,)
    return pl.pallas_call(
        _add_kernel,
        grid=grid,
        in_specs=[
            pl.BlockSpec((BB, P, D), lambda b: (b, 0, 0)),
            pl.BlockSpec((P, D), lambda b: (0, 0)),
        ],
        out_specs=pl.BlockSpec((BB, P, D), lambda b: (b, 0, 0)),
        out_shape=jax.ShapeDtypeStruct((B, P, D), encoded_patches.dtype),
    )(encoded_patches, pos_table)


# speedup vs baseline: 2.4079x; 1.0305x over previous
import jax
import jax.numpy as jnp
from jax.experimental import pallas as pl
from jax.experimental.pallas import tpu as pltpu


def _add_kernel(x_ref, t_ref, o_ref):
    o_ref[...] = x_ref[...] + t_ref[...]


def kernel(encoded_patches, pos_table):
    B, P, D = encoded_patches.shape
    BB = 4
    grid = (B // BB,)
    return pl.pallas_call(
        _add_kernel,
        grid=grid,
        in_specs=[
            pl.BlockSpec((BB, P, D), lambda b: (b, 0, 0)),
            pl.BlockSpec((P, D), lambda b: (0, 0)),
        ],
        out_specs=pl.BlockSpec((BB, P, D), lambda b: (b, 0, 0)),
        out_shape=jax.ShapeDtypeStruct((B, P, D), encoded_patches.dtype),
    )(encoded_patches, pos_table)


# speedup vs baseline: 2.4490x; 1.0171x over previous
import jax
import jax.numpy as jnp
from jax.experimental import pallas as pl
from jax.experimental.pallas import tpu as pltpu


def _add_kernel(x_ref, t_ref, o_ref):
    o_ref[...] = x_ref[...] + t_ref[...]


def kernel(encoded_patches, pos_table):
    B, P, D = encoded_patches.shape
    BB = 8
    grid = (B // BB,)
    return pl.pallas_call(
        _add_kernel,
        grid=grid,
        in_specs=[
            pl.BlockSpec((BB, P, D), lambda b: (b, 0, 0)),
            pl.BlockSpec((P, D), lambda b: (0, 0)),
        ],
        out_specs=pl.BlockSpec((BB, P, D), lambda b: (b, 0, 0)),
        out_shape=jax.ShapeDtypeStruct((B, P, D), encoded_patches.dtype),
    )(encoded_patches, pos_table)
